# baseline (speedup 1.0000x reference)
import functools

import jax
import jax.numpy as jnp
from jax import lax
from jax.experimental import pallas as pl
from jax.experimental.pallas import tpu as pltpu
from jax.experimental.pallas import tpu_sc as plsc

N = 10000
E_RAW = 320000
E_TOT = E_RAW + N
NUM_TYPES = 64
HID = 128
NUM_GRAPHS = 16

NC = 2
NS = 16
HH = HID // NC
K = 128
CHUNKS = 4 * (-(-E_TOT // (NS * K * 4)))
EPT = CHUNKS * K
E_PAD = NS * EPT
N_PAD = 10240
ROWS_PER_TILE = N_PAD // NS
BN = N
GRID = N // BN


def _node1_body(pos_ref, z_ref, emb_ref, w1a_ref, w1b_ref, asw_ref, adw_ref,
                h_ref, as_ref, ad_ref):
    zb = z_ref[0, 0, :]
    oh = (zb[:, None] == lax.broadcasted_iota(jnp.int32, (BN, NUM_TYPES), 1))
    oh = oh.astype(jnp.float32)
    m = jnp.dot(emb_ref[...], w1b_ref[...], preferred_element_type=jnp.float32)
    h = jnp.dot(pos_ref[...], w1a_ref[...], preferred_element_type=jnp.float32)
    h = h + jnp.dot(oh, m, preferred_element_type=jnp.float32)
    h_ref[0] = h[:, :HH].astype(jnp.bfloat16)
    h_ref[1] = h[:, HH:].astype(jnp.bfloat16)
    as_ref[0, 0, :] = jnp.sum(h * asw_ref[...], axis=1)
    ad_ref[0, 0, :] = jnp.sum(h * adw_ref[...], axis=1)


_node1 = pl.pallas_call(
    _node1_body,
    grid=(GRID,),
    in_specs=[
        pl.BlockSpec((BN, 3), lambda i: (i, 0)),
        pl.BlockSpec((1, 1, BN), lambda i: (i, 0, 0)),
        pl.BlockSpec((NUM_TYPES, HID), lambda i: (0, 0)),
        pl.BlockSpec((3, HID), lambda i: (0, 0)),
        pl.BlockSpec((HID, HID), lambda i: (0, 0)),
        pl.BlockSpec((1, HID), lambda i: (0, 0)),
        pl.BlockSpec((1, HID), lambda i: (0, 0)),
    ],
    out_specs=[
        pl.BlockSpec((NC, BN, HH), lambda i: (0, i, 0)),
        pl.BlockSpec((1, 1, BN), lambda i: (i, 0, 0)),
        pl.BlockSpec((1, 1, BN), lambda i: (i, 0, 0)),
    ],
    out_shape=[
        jax.ShapeDtypeStruct((NC, N, HH), jnp.bfloat16),
        jax.ShapeDtypeStruct((GRID, 1, BN), jnp.float32),
        jax.ShapeDtypeStruct((GRID, 1, BN), jnp.float32),
    ],
)


_sc_mesh = plsc.VectorSubcoreMesh(core_axis_name="c", subcore_axis_name="s",
                                  num_cores=NC, num_subcores=NS)


@functools.partial(
    pl.kernel,
    out_type=(
        jax.ShapeDtypeStruct((NC, N_PAD, HH), jnp.bfloat16),
        jax.ShapeDtypeStruct((NC, N_PAD), jnp.float32),
    ),
    mesh=_sc_mesh,
    compiler_params=pltpu.CompilerParams(needs_layout_passes=False,
                                         use_tc_tiling_on_sc=False),
    scratch_types=(
        pltpu.VMEM((N,), jnp.float32),
        pltpu.VMEM((N,), jnp.float32),
        pltpu.VMEM((CHUNKS, K), jnp.int32),
        pltpu.VMEM((CHUNKS, K), jnp.int32),
        pltpu.VMEM((CHUNKS, K), jnp.float32),
        pltpu.VMEM((K, HH), jnp.bfloat16),
        pltpu.VMEM((K, HH), jnp.bfloat16),
        pltpu.VMEM((K, HH), jnp.bfloat16),
        pltpu.VMEM((K, HH), jnp.bfloat16),
        pltpu.VMEM((K, HH), jnp.bfloat16),
        pltpu.VMEM((N_PAD // NS,), jnp.float32),
        pltpu.VMEM_SHARED((N_PAD, HH), jnp.bfloat16),
        pltpu.VMEM_SHARED((N_PAD,), jnp.float32),
        pltpu.SemaphoreType.DMA,
        pltpu.SemaphoreType.DMA,
        pltpu.SemaphoreType.DMA,
        pltpu.SemaphoreType.DMA,
    ),
)
def _sc_edges(h_hbm, as_hbm, ad_hbm, src_hbm, dst_hbm,
              acc_out, den_out,
              as_v, ad_v, src_v, dst_v, w_v, rv0, rv1, rv2, rv3, zbuf,
              zden, acc_s, den_s, sem0, sem1, sem2, sem3):
    cid = lax.axis_index("c")
    sid = lax.axis_index("s")

    pltpu.sync_copy(src_hbm.at[sid], src_v)
    pltpu.sync_copy(dst_hbm.at[sid], dst_v)
    pltpu.sync_copy(as_hbm, as_v)
    pltpu.sync_copy(ad_hbm, ad_v)

    def zb_body(i, carry):
        zbuf[i // (HH // 32), pl.ds((i % (HH // 32)) * 32, 32)] = (
            jnp.zeros((32,), jnp.bfloat16))
        return carry
    lax.fori_loop(0, K * (HH // 32), zb_body, 0)

    def zd_body(i, carry):
        zden[pl.ds(i * 16, 16)] = jnp.zeros((16,), jnp.float32)
        return carry
    lax.fori_loop(0, (N_PAD // NS) // 16, zd_body, 0)

    base_row = sid * ROWS_PER_TILE
    for g in range(ROWS_PER_TILE // K):
        pltpu.sync_copy(zbuf, acc_s.at[pl.ds(base_row + g * K, K)])
    pltpu.sync_copy(zden, den_s.at[pl.ds(base_row, ROWS_PER_TILE)])
    plsc.subcore_barrier()

    ebase = sid * EPT

    def w_body(v, carry):
        j = v // (K // 16)
        k = v % (K // 16)
        s16 = src_v[j, pl.ds(k * 16, 16)]
        d16 = dst_v[j, pl.ds(k * 16, 16)]
        e = plsc.load_gather(as_v, [s16]) + plsc.load_gather(ad_v, [d16])
        e = jnp.where(e >= 0.0, e, 0.2 * e)
        w = jnp.exp(e)
        gid = ebase + v * 16 + lax.iota(jnp.int32, 16)
        w = jnp.where(gid < E_TOT, w, 0.0)
        w_v[j, pl.ds(k * 16, 16)] = w
        return carry
    lax.fori_loop(0, CHUNKS * (K // 16), w_body, 0)

    hc = h_hbm.at[cid]

    def _scale(rv, j):
        def e_body(kk, c2):
            w16 = w_v[j, pl.ds(kk * 16, 16)]
            for e in range(16):
                wf = jnp.full((16,), w16[e], jnp.float32)
                wvb = plsc.pack(wf, wf, format=plsc.PackFormat.INTERLEAVED)
                row = kk * 16 + e
                for g in range(HH // 32):
                    rv[row, pl.ds(g * 32, 32)] = (
                        rv[row, pl.ds(g * 32, 32)] * wvb)
            return c2
        lax.fori_loop(0, K // 16, e_body, 0)

    def _den(j):
        pltpu.sync_copy(w_v.at[j], den_s.at[dst_v.at[j]], add=True)

    bufs = (rv0, rv1, rv2, rv3)
    sems = (sem0, sem1, sem2, sem3)
    for b in range(3):
        pltpu.async_copy(hc.at[src_v.at[b]], bufs[b], sems[b])

    def f_body(i, carry):
        j0 = 4 * i
        for b in range(4):
            j = j0 + b

            @pl.when(j + 3 < CHUNKS)
            def _():
                pltpu.async_copy(hc.at[src_v.at[j + 3]],
                                 bufs[(b + 3) % 4], sems[(b + 3) % 4])
            pltpu.make_async_copy(hc.at[src_v.at[j]], bufs[b], sems[b]).wait()
            _scale(bufs[b], j)
            pltpu.sync_copy(bufs[b], acc_s.at[dst_v.at[j]], add=True)
            _den(j)
        return carry
    lax.fori_loop(0, CHUNKS // 4, f_body, 0)

    plsc.subcore_barrier()
    pltpu.sync_copy(acc_s.at[pl.ds(base_row, ROWS_PER_TILE)],
                    acc_out.at[cid, pl.ds(base_row, ROWS_PER_TILE)])
    pltpu.sync_copy(den_s.at[pl.ds(base_row, ROWS_PER_TILE)],
                    den_out.at[cid, pl.ds(base_row, ROWS_PER_TILE)])


def _mid_body(acc_ref, den_ref, b_ref, w_ref, asw_ref, adw_ref,
              h_ref, as_ref, ad_ref):
    a = jnp.concatenate([acc_ref[0], acc_ref[1]], axis=-1).astype(jnp.float32)
    dn = den_ref[0, 0, 0, :] + 1e-16
    x = a / dn[:, None] + b_ref[...]
    x = jnp.where(x > 0.0, x, jnp.exp(x) - 1.0)
    h = jnp.dot(x, w_ref[...], preferred_element_type=jnp.float32)
    h_ref[0] = h[:, :HH].astype(jnp.bfloat16)
    h_ref[1] = h[:, HH:].astype(jnp.bfloat16)
    as_ref[0, 0, :] = jnp.sum(h * asw_ref[...], axis=1)
    ad_ref[0, 0, :] = jnp.sum(h * adw_ref[...], axis=1)


_mid = pl.pallas_call(
    _mid_body,
    grid=(GRID,),
    in_specs=[
        pl.BlockSpec((NC, BN, HH), lambda i: (0, i, 0)),
        pl.BlockSpec((NC, 1, 1, BN), lambda i: (0, i, 0, 0)),
        pl.BlockSpec((1, HID), lambda i: (0, 0)),
        pl.BlockSpec((HID, HID), lambda i: (0, 0)),
        pl.BlockSpec((1, HID), lambda i: (0, 0)),
        pl.BlockSpec((1, HID), lambda i: (0, 0)),
    ],
    out_specs=[
        pl.BlockSpec((NC, BN, HH), lambda i: (0, i, 0)),
        pl.BlockSpec((1, 1, BN), lambda i: (i, 0, 0)),
        pl.BlockSpec((1, 1, BN), lambda i: (i, 0, 0)),
    ],
    out_shape=[
        jax.ShapeDtypeStruct((NC, N, HH), jnp.bfloat16),
        jax.ShapeDtypeStruct((GRID, 1, BN), jnp.float32),
        jax.ShapeDtypeStruct((GRID, 1, BN), jnp.float32),
    ],
)


def _final_body(acc_ref, den_ref, b_ref, wl_ref, bl_ref, batch_ref,
                out_ref, ssum, scnt):
    i = pl.program_id(0)

    @pl.when(i == 0)
    def _():
        ssum[...] = jnp.zeros((NUM_GRAPHS, HID), jnp.float32)
        scnt[...] = jnp.zeros((NUM_GRAPHS, HID), jnp.float32)

    a = jnp.concatenate([acc_ref[0], acc_ref[1]], axis=-1).astype(jnp.float32)
    dn = den_ref[0, 0, 0, :] + 1e-16
    x = a / dn[:, None] + b_ref[...]
    x = jnp.where(x > 0.0, x, jnp.exp(x) - 1.0)
    y = jnp.dot(x, wl_ref[...], preferred_element_type=jnp.float32) + bl_ref[...]
    bb = batch_ref[0, 0, :]
    oh = (bb[:, None] == lax.broadcasted_iota(jnp.int32, (BN, NUM_GRAPHS), 1))
    oh = oh.astype(jnp.float32)
    ssum[...] += lax.dot_general(oh, y, (((0,), (0,)), ((), ())),
                                 preferred_element_type=jnp.float32)
    scnt[...] += lax.dot_general(oh, jnp.ones((BN, HID), jnp.float32),
                                 (((0,), (0,)), ((), ())),
                                 preferred_element_type=jnp.float32)

    @pl.when(i == GRID - 1)
    def _():
        out_ref[...] = ssum[...] / jnp.maximum(scnt[...], 1.0)


_final = pl.pallas_call(
    _final_body,
    grid=(GRID,),
    in_specs=[
        pl.BlockSpec((NC, BN, HH), lambda i: (0, i, 0)),
        pl.BlockSpec((NC, 1, 1, BN), lambda i: (0, i, 0, 0)),
        pl.BlockSpec((1, HID), lambda i: (0, 0)),
        pl.BlockSpec((HID, HID), lambda i: (0, 0)),
        pl.BlockSpec((1, HID), lambda i: (0, 0)),
        pl.BlockSpec((1, 1, BN), lambda i: (i, 0, 0)),
    ],
    out_specs=pl.BlockSpec((NUM_GRAPHS, HID), lambda i: (0, 0)),
    out_shape=jax.ShapeDtypeStruct((NUM_GRAPHS, HID), jnp.float32),
    scratch_shapes=[
        pltpu.VMEM((NUM_GRAPHS, HID), jnp.float32),
        pltpu.VMEM((NUM_GRAPHS, HID), jnp.float32),
    ],
)


def kernel(pos, z, edge_index, batch, emb, W1, a_s1, a_d1, b1,
           W2, a_s2, a_d2, b2, Wl, bl):
    z3 = z.astype(jnp.int32).reshape(GRID, 1, BN)
    batch3 = batch.astype(jnp.int32).reshape(GRID, 1, BN)
    loop = jnp.arange(N, dtype=jnp.int32)
    src = jnp.concatenate([edge_index[0].astype(jnp.int32), loop])
    dst = jnp.concatenate([edge_index[1].astype(jnp.int32), loop])
    pad = E_PAD - E_TOT
    src = jnp.concatenate([src, jnp.zeros((pad,), jnp.int32)]).reshape(NS, CHUNKS, K)
    dst = jnp.concatenate([dst, jnp.zeros((pad,), jnp.int32)]).reshape(NS, CHUNKS, K)

    h1, as1, ad1 = _node1(pos, z3, emb, W1[:3], W1[3:],
                          a_s1.reshape(1, HID), a_d1.reshape(1, HID))
    acc1, den1 = _sc_edges(h1, as1.reshape(N), ad1.reshape(N), src, dst)
    h2, as2, ad2 = _mid(acc1[:, :N],
                        den1[:1, :N].reshape(1, GRID, 1, BN),
                        b1.reshape(1, HID), W2,
                        a_s2.reshape(1, HID), a_d2.reshape(1, HID))
    acc2, den2 = _sc_edges(h2, as2.reshape(N), ad2.reshape(N), src, dst)
    out = _final(acc2[:, :N],
                 den2[:, :N].reshape(NC, GRID, 1, BN),
                 b2.reshape(1, HID), Wl, bl.reshape(1, HID), batch3)
    return out


# speedup vs baseline: 1.1738x; 1.1738x over previous
import functools

import jax
import jax.numpy as jnp
from jax import lax
from jax.experimental import pallas as pl
from jax.experimental.pallas import tpu as pltpu
from jax.experimental.pallas import tpu_sc as plsc

N = 10000
E_RAW = 320000
E_TOT = E_RAW + N
NUM_TYPES = 64
HID = 128
NUM_GRAPHS = 16

NC = 2
NS = 16
HH = HID // NC
K = 128
CHUNKS = -(-E_TOT // (NS * K))
EPT = CHUNKS * K
E_PAD = NS * EPT
N_PAD = 10240
ROWS_PER_TILE = N_PAD // NS
BN = N
GRID = N // BN


def _node1_body(pos_ref, z_ref, emb_ref, w1a_ref, w1b_ref, asw_ref, adw_ref,
                h_ref, as_ref, ad_ref):
    zb = z_ref[0, 0, :]
    oh = (zb[:, None] == lax.broadcasted_iota(jnp.int32, (BN, NUM_TYPES), 1))
    oh = oh.astype(jnp.float32)
    m = jnp.dot(emb_ref[...], w1b_ref[...], preferred_element_type=jnp.float32)
    h = jnp.dot(pos_ref[...], w1a_ref[...], preferred_element_type=jnp.float32)
    h = h + jnp.dot(oh, m, preferred_element_type=jnp.float32)
    h_ref[0] = h[:, :HH].astype(jnp.bfloat16)
    h_ref[1] = h[:, HH:].astype(jnp.bfloat16)
    as_ref[0, 0, :] = jnp.sum(h * asw_ref[...], axis=1)
    ad_ref[0, 0, :] = jnp.sum(h * adw_ref[...], axis=1)


_node1 = pl.pallas_call(
    _node1_body,
    grid=(GRID,),
    in_specs=[
        pl.BlockSpec((BN, 3), lambda i: (i, 0)),
        pl.BlockSpec((1, 1, BN), lambda i: (i, 0, 0)),
        pl.BlockSpec((NUM_TYPES, HID), lambda i: (0, 0)),
        pl.BlockSpec((3, HID), lambda i: (0, 0)),
        pl.BlockSpec((HID, HID), lambda i: (0, 0)),
        pl.BlockSpec((1, HID), lambda i: (0, 0)),
        pl.BlockSpec((1, HID), lambda i: (0, 0)),
    ],
    out_specs=[
        pl.BlockSpec((NC, BN, HH), lambda i: (0, i, 0)),
        pl.BlockSpec((1, 1, BN), lambda i: (i, 0, 0)),
        pl.BlockSpec((1, 1, BN), lambda i: (i, 0, 0)),
    ],
    out_shape=[
        jax.ShapeDtypeStruct((NC, N, HH), jnp.bfloat16),
        jax.ShapeDtypeStruct((GRID, 1, BN), jnp.float32),
        jax.ShapeDtypeStruct((GRID, 1, BN), jnp.float32),
    ],
)


_sc_mesh = plsc.VectorSubcoreMesh(core_axis_name="c", subcore_axis_name="s",
                                  num_cores=NC, num_subcores=NS)


@functools.partial(
    pl.kernel,
    out_type=(
        jax.ShapeDtypeStruct((NC, N_PAD, HH), jnp.bfloat16),
        jax.ShapeDtypeStruct((NC, N_PAD), jnp.float32),
    ),
    mesh=_sc_mesh,
    compiler_params=pltpu.CompilerParams(needs_layout_passes=False,
                                         use_tc_tiling_on_sc=False),
    scratch_types=(
        pltpu.VMEM((N,), jnp.float32),
        pltpu.VMEM((N,), jnp.float32),
        pltpu.VMEM((CHUNKS, K), jnp.int32),
        pltpu.VMEM((CHUNKS, K), jnp.int32),
        pltpu.VMEM((CHUNKS, K), jnp.float32),
        pltpu.VMEM((K, HH), jnp.bfloat16),
        pltpu.VMEM((K, HH), jnp.bfloat16),
        pltpu.VMEM((K, HH), jnp.bfloat16),
        pltpu.VMEM((N_PAD // NS,), jnp.float32),
        pltpu.VMEM_SHARED((N_PAD, HH), jnp.bfloat16),
        pltpu.VMEM_SHARED((N_PAD,), jnp.float32),
        pltpu.SemaphoreType.DMA,
        pltpu.SemaphoreType.DMA,
    ),
)
def _sc_edges(h_hbm, as_hbm, ad_hbm, src_hbm, dst_hbm,
              acc_out, den_out,
              as_v, ad_v, src_v, dst_v, w_v, rv0, rv1, zbuf, zden,
              acc_s, den_s, sem0, sem1):
    cid = lax.axis_index("c")
    sid = lax.axis_index("s")

    pltpu.sync_copy(src_hbm.at[sid], src_v)
    pltpu.sync_copy(dst_hbm.at[sid], dst_v)
    pltpu.sync_copy(as_hbm, as_v)
    pltpu.sync_copy(ad_hbm, ad_v)

    def zb_body(i, carry):
        zbuf[i // (HH // 32), pl.ds((i % (HH // 32)) * 32, 32)] = (
            jnp.zeros((32,), jnp.bfloat16))
        return carry
    lax.fori_loop(0, K * (HH // 32), zb_body, 0)

    def zd_body(i, carry):
        zden[pl.ds(i * 16, 16)] = jnp.zeros((16,), jnp.float32)
        return carry
    lax.fori_loop(0, (N_PAD // NS) // 16, zd_body, 0)

    base_row = sid * ROWS_PER_TILE
    for g in range(ROWS_PER_TILE // K):
        pltpu.sync_copy(zbuf, acc_s.at[pl.ds(base_row + g * K, K)])
    pltpu.sync_copy(zden, den_s.at[pl.ds(base_row, ROWS_PER_TILE)])
    plsc.subcore_barrier()

    ebase = sid * EPT

    def w_body(v, carry):
        j = v // (K // 16)
        k = v % (K // 16)
        s16 = src_v[j, pl.ds(k * 16, 16)]
        d16 = dst_v[j, pl.ds(k * 16, 16)]
        e = plsc.load_gather(as_v, [s16]) + plsc.load_gather(ad_v, [d16])
        e = jnp.where(e >= 0.0, e, 0.2 * e)
        w = jnp.exp(e)
        gid = ebase + v * 16 + lax.iota(jnp.int32, 16)
        w = jnp.where(gid < E_TOT, w, 0.0)
        w_v[j, pl.ds(k * 16, 16)] = w
        return carry
    lax.fori_loop(0, CHUNKS * (K // 16), w_body, 0)

    hc = h_hbm.at[cid]

    def _scale(rv, j):
        def e_body(kk, c2):
            w16 = w_v[j, pl.ds(kk * 16, 16)]
            for e in range(16):
                wf = jnp.full((16,), w16[e], jnp.float32)
                wvb = plsc.pack(wf, wf, format=plsc.PackFormat.INTERLEAVED)
                row = kk * 16 + e
                for g in range(HH // 32):
                    rv[row, pl.ds(g * 32, 32)] = (
                        rv[row, pl.ds(g * 32, 32)] * wvb)
            return c2
        lax.fori_loop(0, K // 16, e_body, 0)

    def _den(j):
        pltpu.sync_copy(w_v.at[j], den_s.at[dst_v.at[j]], add=True)

    pltpu.async_copy(hc.at[src_v.at[0]], rv0, sem0)

    def f_body(i, carry):
        j0 = 2 * i
        pltpu.async_copy(hc.at[src_v.at[j0 + 1]], rv1, sem1)
        pltpu.make_async_copy(hc.at[src_v.at[j0]], rv0, sem0).wait()
        _scale(rv0, j0)
        pltpu.sync_copy(rv0, acc_s.at[dst_v.at[j0]], add=True)
        _den(j0)

        @pl.when(j0 + 2 < CHUNKS)
        def _():
            pltpu.async_copy(hc.at[src_v.at[j0 + 2]], rv0, sem0)
        pltpu.make_async_copy(hc.at[src_v.at[j0 + 1]], rv1, sem1).wait()
        _scale(rv1, j0 + 1)
        pltpu.sync_copy(rv1, acc_s.at[dst_v.at[j0 + 1]], add=True)
        _den(j0 + 1)
        return carry
    lax.fori_loop(0, CHUNKS // 2, f_body, 0)

    plsc.subcore_barrier()
    pltpu.sync_copy(acc_s.at[pl.ds(base_row, ROWS_PER_TILE)],
                    acc_out.at[cid, pl.ds(base_row, ROWS_PER_TILE)])
    pltpu.sync_copy(den_s.at[pl.ds(base_row, ROWS_PER_TILE)],
                    den_out.at[cid, pl.ds(base_row, ROWS_PER_TILE)])


def _mid_body(acc_ref, den_ref, b_ref, w_ref, asw_ref, adw_ref,
              h_ref, as_ref, ad_ref):
    a = jnp.concatenate([acc_ref[0], acc_ref[1]], axis=-1).astype(jnp.float32)
    dn = den_ref[0, 0, 0, :] + 1e-16
    x = a / dn[:, None] + b_ref[...]
    x = jnp.where(x > 0.0, x, jnp.exp(x) - 1.0)
    h = jnp.dot(x, w_ref[...], preferred_element_type=jnp.float32)
    h_ref[0] = h[:, :HH].astype(jnp.bfloat16)
    h_ref[1] = h[:, HH:].astype(jnp.bfloat16)
    as_ref[0, 0, :] = jnp.sum(h * asw_ref[...], axis=1)
    ad_ref[0, 0, :] = jnp.sum(h * adw_ref[...], axis=1)


_mid = pl.pallas_call(
    _mid_body,
    grid=(GRID,),
    in_specs=[
        pl.BlockSpec((NC, BN, HH), lambda i: (0, i, 0)),
        pl.BlockSpec((NC, 1, 1, BN), lambda i: (0, i, 0, 0)),
        pl.BlockSpec((1, HID), lambda i: (0, 0)),
        pl.BlockSpec((HID, HID), lambda i: (0, 0)),
        pl.BlockSpec((1, HID), lambda i: (0, 0)),
        pl.BlockSpec((1, HID), lambda i: (0, 0)),
    ],
    out_specs=[
        pl.BlockSpec((NC, BN, HH), lambda i: (0, i, 0)),
        pl.BlockSpec((1, 1, BN), lambda i: (i, 0, 0)),
        pl.BlockSpec((1, 1, BN), lambda i: (i, 0, 0)),
    ],
    out_shape=[
        jax.ShapeDtypeStruct((NC, N, HH), jnp.bfloat16),
        jax.ShapeDtypeStruct((GRID, 1, BN), jnp.float32),
        jax.ShapeDtypeStruct((GRID, 1, BN), jnp.float32),
    ],
)


def _final_body(acc_ref, den_ref, b_ref, wl_ref, bl_ref, batch_ref,
                out_ref, ssum, scnt):
    i = pl.program_id(0)

    @pl.when(i == 0)
    def _():
        ssum[...] = jnp.zeros((NUM_GRAPHS, HID), jnp.float32)
        scnt[...] = jnp.zeros((NUM_GRAPHS, HID), jnp.float32)

    a = jnp.concatenate([acc_ref[0], acc_ref[1]], axis=-1).astype(jnp.float32)
    dn = den_ref[0, 0, 0, :] + 1e-16
    x = a / dn[:, None] + b_ref[...]
    x = jnp.where(x > 0.0, x, jnp.exp(x) - 1.0)
    y = jnp.dot(x, wl_ref[...], preferred_element_type=jnp.float32) + bl_ref[...]
    bb = batch_ref[0, 0, :]
    oh = (bb[:, None] == lax.broadcasted_iota(jnp.int32, (BN, NUM_GRAPHS), 1))
    oh = oh.astype(jnp.float32)
    ssum[...] += lax.dot_general(oh, y, (((0,), (0,)), ((), ())),
                                 preferred_element_type=jnp.float32)
    scnt[...] += lax.dot_general(oh, jnp.ones((BN, HID), jnp.float32),
                                 (((0,), (0,)), ((), ())),
                                 preferred_element_type=jnp.float32)

    @pl.when(i == GRID - 1)
    def _():
        out_ref[...] = ssum[...] / jnp.maximum(scnt[...], 1.0)


_final = pl.pallas_call(
    _final_body,
    grid=(GRID,),
    in_specs=[
        pl.BlockSpec((NC, BN, HH), lambda i: (0, i, 0)),
        pl.BlockSpec((NC, 1, 1, BN), lambda i: (0, i, 0, 0)),
        pl.BlockSpec((1, HID), lambda i: (0, 0)),
        pl.BlockSpec((HID, HID), lambda i: (0, 0)),
        pl.BlockSpec((1, HID), lambda i: (0, 0)),
        pl.BlockSpec((1, 1, BN), lambda i: (i, 0, 0)),
    ],
    out_specs=pl.BlockSpec((NUM_GRAPHS, HID), lambda i: (0, 0)),
    out_shape=jax.ShapeDtypeStruct((NUM_GRAPHS, HID), jnp.float32),
    scratch_shapes=[
        pltpu.VMEM((NUM_GRAPHS, HID), jnp.float32),
        pltpu.VMEM((NUM_GRAPHS, HID), jnp.float32),
    ],
)


def kernel(pos, z, edge_index, batch, emb, W1, a_s1, a_d1, b1,
           W2, a_s2, a_d2, b2, Wl, bl):
    z3 = z.astype(jnp.int32).reshape(GRID, 1, BN)
    batch3 = batch.astype(jnp.int32).reshape(GRID, 1, BN)
    loop = jnp.arange(N, dtype=jnp.int32)
    src = jnp.concatenate([edge_index[0].astype(jnp.int32), loop])
    dst = jnp.concatenate([edge_index[1].astype(jnp.int32), loop])
    pad = E_PAD - E_TOT
    src = jnp.concatenate([src, jnp.zeros((pad,), jnp.int32)]).reshape(NS, CHUNKS, K)
    dst = jnp.concatenate([dst, jnp.zeros((pad,), jnp.int32)]).reshape(NS, CHUNKS, K)

    h1, as1, ad1 = _node1(pos, z3, emb, W1[:3], W1[3:],
                          a_s1.reshape(1, HID), a_d1.reshape(1, HID))
    acc1, den1 = _sc_edges(h1, as1.reshape(N), ad1.reshape(N), src, dst)
    h2, as2, ad2 = _mid(acc1[:, :N],
                        den1[:1, :N].reshape(1, GRID, 1, BN),
                        b1.reshape(1, HID), W2,
                        a_s2.reshape(1, HID), a_d2.reshape(1, HID))
    acc2, den2 = _sc_edges(h2, as2.reshape(N), ad2.reshape(N), src, dst)
    out = _final(acc2,
                 den2[:, :N].reshape(NC, GRID, 1, BN),
                 b2.reshape(1, HID), Wl, bl.reshape(1, HID), batch3)
    return out
